# baseline (speedup 1.0000x reference)
import functools

import jax
import jax.numpy as jnp
from jax import lax
from jax.experimental import pallas as pl
from jax.experimental.pallas import tpu as pltpu

NUM_EMBEDDINGS = 1000000
EMBED_DIM = 64
BATCH = 16384
NSEM = 8


def _make_tc(n_rows, nsem=8, unroll=8, wave=8192):
    n_waves = n_rows // wave
    groups = wave // nsem

    def body(idx_s, w_hbm, out_hbm, buf, sems):
        for wv in range(n_waves):
            w0 = wv * wave

            def issue(o, _):
                for j in range(nsem):
                    i = w0 + o * nsem + j
                    r = idx_s[i]
                    pltpu.make_async_copy(
                        w_hbm.at[pl.ds(r, 1)],
                        buf.at[pl.ds(i, 1)],
                        sems.at[j],
                    ).start()
                return 0

            lax.fori_loop(0, groups, issue, 0, unroll=unroll)
            for j in range(nsem):
                pltpu.make_async_copy(
                    w_hbm.at[pl.ds(0, groups)],
                    buf.at[pl.ds(w0 + j * groups, groups)],
                    sems.at[j],
                ).wait()
        pltpu.sync_copy(buf, out_hbm)

    return pl.pallas_call(
        body,
        out_shape=jax.ShapeDtypeStruct((n_rows, EMBED_DIM), jnp.float32),
        in_specs=[
            pl.BlockSpec(memory_space=pltpu.SMEM),
            pl.BlockSpec(memory_space=pl.ANY),
        ],
        out_specs=pl.BlockSpec(memory_space=pl.ANY),
        scratch_shapes=[
            pltpu.VMEM((n_rows, EMBED_DIM), jnp.float32),
            pltpu.SemaphoreType.DMA((nsem,)),
        ],
    )


_gather_tc = _make_tc(BATCH)


def kernel(batch, w):
    return _gather_tc(batch.astype(jnp.int32), w)


# speedup vs baseline: 1.1403x; 1.1403x over previous
import functools

import jax
import jax.numpy as jnp
from jax import lax
from jax.experimental import pallas as pl
from jax.experimental.pallas import tpu as pltpu
from jax.experimental.pallas import tpu_sc as plsc

NUM_EMBEDDINGS = 1000000
EMBED_DIM = 64
BATCH = 16384
NUM_CORES = 2
NUM_SUBCORES = 16
NUM_WORKERS = NUM_CORES * NUM_SUBCORES
B_PER_W = BATCH // NUM_WORKERS
LANES = 16
NSEM = 4
Q = B_PER_W // NSEM

_mesh = plsc.VectorSubcoreMesh(core_axis_name="c", subcore_axis_name="s")


@functools.partial(
    pl.kernel,
    mesh=_mesh,
    out_type=jax.ShapeDtypeStruct((BATCH, EMBED_DIM), jnp.float32),
    scratch_types=[
        pltpu.VMEM((B_PER_W,), jnp.int32),
        pltpu.VMEM((B_PER_W, EMBED_DIM), jnp.float32),
        pltpu.SemaphoreType.DMA((NSEM,)),
    ],
)
def _gather_sc(idx_hbm, table_hbm, out_hbm, idx_v, rows_v, sems):
    wid = lax.axis_index("s") * NUM_CORES + lax.axis_index("c")
    base = wid * B_PER_W
    pltpu.sync_copy(idx_hbm.at[pl.ds(base, B_PER_W)], idx_v)

    @pl.loop(0, B_PER_W // LANES)
    def _group(g):
        vec = idx_v[pl.ds(g * LANES, LANES)]
        for j in range(LANES):
            r = vec[j]
            i = g * LANES + j
            pltpu.make_async_copy(
                table_hbm.at[pl.ds(r, 1)],
                rows_v.at[pl.ds(i, 1)],
                sems.at[j % NSEM],
            ).start()

    for q in range(NSEM):
        pltpu.make_async_copy(
            table_hbm.at[pl.ds(0, Q)],
            rows_v.at[pl.ds(q * Q, Q)],
            sems.at[q],
        ).wait()
    pltpu.sync_copy(rows_v, out_hbm.at[pl.ds(base, B_PER_W)])


def kernel(batch, w):
    return _gather_sc(batch.astype(jnp.int32), w)
